# Initial kernel scaffold; baseline (speedup 1.0000x reference)
import jax
import jax.numpy as jnp
from jax.experimental import pallas as pl


def kernel(token_ids, weights):
    raise NotImplementedError("write your pallas kernel here")


# speedup vs baseline: 1.0229x; 1.0229x over previous
import functools

import jax
import jax.numpy as jnp
from jax import lax
from jax.experimental import pallas as pl
from jax.experimental.pallas import tpu as pltpu
from jax.experimental.pallas import tpu_sc as plsc

NUM_WORKERS = 32
GROUP = 128


def _build(B, D, n_groups):
    mesh = plsc.VectorSubcoreMesh(core_axis_name="c", subcore_axis_name="s")

    @functools.partial(
        pl.kernel,
        mesh=mesh,
        out_type=jax.ShapeDtypeStruct((B, D), jnp.float32),
        compiler_params=pltpu.CompilerParams(use_tc_tiling_on_sc=False),
        scratch_types=[
            pltpu.VMEM((n_groups, GROUP), jnp.int32),
            pltpu.VMEM((GROUP, D), jnp.float32),
            pltpu.SemaphoreType.DMA,
        ],
    )
    def body(idx_hbm, table_hbm, out_hbm, idx_v, rows_v, sem):
        wid = lax.axis_index("s") * 2 + lax.axis_index("c")
        gbase = wid * n_groups
        pltpu.sync_copy(idx_hbm.at[pl.ds(gbase, n_groups)], idx_v)

        def gbody(g, carry):
            pltpu.async_copy(table_hbm.at[idx_v.at[g]], rows_v, sem).wait()
            pltpu.sync_copy(rows_v, out_hbm.at[pl.ds((gbase + g) * GROUP, GROUP)])
            return carry

        lax.fori_loop(0, n_groups, gbody, 0)

    return body


def kernel(token_ids, weights):
    B0, S = token_ids.shape
    V, D = weights.shape
    B = B0 * S
    n_groups = B // (NUM_WORKERS * GROUP)
    idx = token_ids.reshape(NUM_WORKERS * n_groups, GROUP).astype(jnp.int32)
    out = _build(B, D, n_groups)(idx, weights)
    return out.reshape(B0, S, D)


# speedup vs baseline: 1.1136x; 1.0886x over previous
import functools

import jax
import jax.numpy as jnp
from jax import lax
from jax.experimental import pallas as pl
from jax.experimental.pallas import tpu as pltpu
from jax.experimental.pallas import tpu_sc as plsc

NUM_WORKERS = 32
GROUP = 128
K = 10
CHUNK = K * GROUP


def _build(B, D, n_groups):
    n_chunks = n_groups // K
    mesh = plsc.VectorSubcoreMesh(core_axis_name="c", subcore_axis_name="s")

    @functools.partial(
        pl.kernel,
        mesh=mesh,
        out_type=jax.ShapeDtypeStruct((B, D), jnp.float32),
        compiler_params=pltpu.CompilerParams(use_tc_tiling_on_sc=False),
        scratch_types=[
            pltpu.VMEM((n_groups, GROUP), jnp.int32),
            pltpu.VMEM((2, CHUNK, D), jnp.float32),
            pltpu.SemaphoreType.DMA,
            pltpu.SemaphoreType.DMA,
            pltpu.SemaphoreType.DMA,
            pltpu.SemaphoreType.DMA,
        ],
    )
    def body(idx_hbm, table_hbm, out_hbm, idx_v, rows_v, g0, g1, w0, w1):
        wid = lax.axis_index("s") * 2 + lax.axis_index("c")
        gbase = wid * n_groups
        rbase = gbase * GROUP
        pltpu.sync_copy(idx_hbm.at[pl.ds(gbase, n_groups)], idx_v)
        gsem = (g0, g1)
        wsem = (w0, w1)

        def fire(c, slot):
            for j in range(K):
                pltpu.async_copy(
                    table_hbm.at[idx_v.at[c * K + j]],
                    rows_v.at[slot].at[pl.ds(j * GROUP, GROUP)],
                    gsem[slot],
                )

        def drain_gather(slot):
            pltpu.make_async_copy(
                out_hbm.at[pl.ds(0, CHUNK)], rows_v.at[slot], gsem[slot]
            ).wait()

        def drain_wb(slot):
            pltpu.make_async_copy(
                rows_v.at[slot], out_hbm.at[pl.ds(0, CHUNK)], wsem[slot]
            ).wait()

        fire(0, 0)

        def outer(c, carry):
            for slot in (0, 1):
                cc = c + slot
                nxt = cc + 1

                @pl.when(nxt < n_chunks)
                def _():
                    @pl.when(cc >= 1)
                    def _():
                        drain_wb(1 - slot)
                    fire(nxt, 1 - slot)

                drain_gather(slot)
                pltpu.async_copy(
                    rows_v.at[slot],
                    out_hbm.at[pl.ds(rbase + cc * CHUNK, CHUNK)],
                    wsem[slot],
                )
            return carry

        lax.fori_loop(0, n_chunks // 2, lambda i, cr: outer(i * 2, cr), 0)
        drain_wb(0)
        drain_wb(1)

    return body


def kernel(token_ids, weights):
    B0, S = token_ids.shape
    V, D = weights.shape
    B = B0 * S
    n_groups = B // (NUM_WORKERS * GROUP)
    idx = token_ids.reshape(NUM_WORKERS * n_groups, GROUP).astype(jnp.int32)
    out = _build(B, D, n_groups)(idx, weights)
    return out.reshape(B0, S, D)


# speedup vs baseline: 1.6065x; 1.4427x over previous
import functools

import jax
import jax.numpy as jnp
from jax import lax
from jax.experimental import pallas as pl
from jax.experimental.pallas import tpu as pltpu
from jax.experimental.pallas import tpu_sc as plsc

NW = 32
SB = 512
TB = 128
NH = SB // TB


def _build(S, D, Bc):
    Sp = S + (-S) % 8
    NT = S * NH
    mesh = plsc.VectorSubcoreMesh(core_axis_name="c", subcore_axis_name="s")

    @functools.partial(
        pl.kernel,
        mesh=mesh,
        out_type=jax.ShapeDtypeStruct((S, D, Bc), jnp.float32),
        compiler_params=pltpu.CompilerParams(
            use_tc_tiling_on_sc=True, needs_layout_passes=False
        ),
        scratch_types=[
            pltpu.VMEM((Sp, SB), jnp.int32),
            pltpu.VMEM((2 * TB, 128), jnp.float32),
            pltpu.VMEM((2 * D, TB), jnp.float32),
            pltpu.SemaphoreType.DMA,
            pltpu.SemaphoreType.DMA,
            pltpu.SemaphoreType.DMA,
            pltpu.SemaphoreType.DMA,
        ],
    )
    def body(idx_hbm, w128_hbm, out_hbm, idx_v, gbuf, tbuf, g0, g1, w0, w1):
        wid = lax.axis_index("s") * 2 + lax.axis_index("c")
        b0w = wid * SB
        gsem = (g0, g1)
        wsem = (w0, w1)

        pltpu.sync_copy(idx_hbm.at[:, pl.ds(b0w, SB)], idx_v)

        iota = lax.iota(jnp.int32, 16)

        def fire(c, slot):
            s_ = c // NH
            boff = (c % NH) * TB
            pltpu.async_copy(
                w128_hbm.at[idx_v.at[s_, pl.ds(boff, TB)]],
                gbuf.at[pl.ds(slot * TB, TB)],
                gsem[slot],
            )

        def drain_gather(slot):
            pltpu.make_async_copy(
                w128_hbm.at[pl.ds(0, TB)],
                gbuf.at[pl.ds(slot * TB, TB)],
                gsem[slot],
            ).wait()

        def transpose(slot):
            gv = gbuf.at[pl.ds(slot * TB, TB)]
            vjs = [g * 16 + iota for g in range(TB // 16)]

            def dbody(d, carry):
                vd = iota * 0 + d
                for g in range(TB // 16):
                    v = plsc.load_gather(gv, [vjs[g], vd])
                    tbuf[slot * D + d, pl.ds(g * 16, 16)] = v
                return carry

            lax.fori_loop(0, D, dbody, 0)

        def writeback(c, slot):
            s_ = c // NH
            boff = (c % NH) * TB
            pltpu.async_copy(
                tbuf.at[pl.ds(slot * D, D)],
                out_hbm.at[s_, :, pl.ds(b0w + boff, TB)],
                wsem[slot],
            )

        def drain_wb(slot):
            pltpu.make_async_copy(
                tbuf.at[pl.ds(slot * D, D)],
                out_hbm.at[0, :, pl.ds(0, TB)],
                wsem[slot],
            ).wait()

        fire(0, 0)

        def outer(c, carry):
            for slot in (0, 1):
                cc = c + slot
                nxt = cc + 1

                @pl.when(nxt < NT)
                def _():
                    @pl.when(cc >= 1)
                    def _():
                        drain_wb(1 - slot)
                    fire(nxt, 1 - slot)

                drain_gather(slot)
                transpose(slot)
                writeback(cc, slot)
            return carry

        lax.fori_loop(0, NT // 2, lambda i, cr: outer(i * 2, cr), 0)
        drain_wb(0)
        drain_wb(1)

    return body


def kernel(token_ids, weights):
    B0, S = token_ids.shape
    V, D = weights.shape
    idxp = jnp.pad(token_ids.T.astype(jnp.int32), ((0, (-S) % 8), (0, 0)))
    w128 = jnp.pad(weights, ((0, 0), (0, 128 - D)))
    oT = _build(S, D, B0)(idxp, w128)
    return oT.transpose(2, 0, 1)


# speedup vs baseline: 1.6339x; 1.0171x over previous
import functools

import jax
import jax.numpy as jnp
from jax import lax
from jax.experimental import pallas as pl
from jax.experimental.pallas import tpu as pltpu
from jax.experimental.pallas import tpu_sc as plsc

NW = 32
SB = 512
TB = 128
NH = SB // TB
NS = 4


def _build(S, D, Bc):
    Sp = S + (-S) % 8
    NT = S * NH
    mesh = plsc.VectorSubcoreMesh(core_axis_name="c", subcore_axis_name="s")

    @functools.partial(
        pl.kernel,
        mesh=mesh,
        out_type=jax.ShapeDtypeStruct((S, D, Bc), jnp.float32),
        compiler_params=pltpu.CompilerParams(
            use_tc_tiling_on_sc=True, needs_layout_passes=False
        ),
        scratch_types=[
            pltpu.VMEM((Sp, SB), jnp.int32),
            pltpu.VMEM((NS, TB), jnp.int32),
            pltpu.VMEM((NS * TB,), jnp.int32),
            pltpu.VMEM((NS * TB, 128), jnp.float32),
            pltpu.VMEM((NS * D, TB), jnp.float32),
        ] + [pltpu.SemaphoreType.DMA] * (2 * NS),
    )
    def body(idx_hbm, w4_hbm, out_hbm, idx_v, idx4_v, off_v, gbuf, tbuf,
             *sems):
        gsem = sems[:NS]
        wsem = sems[NS:]
        wid = lax.axis_index("s") * 2 + lax.axis_index("c")
        b0w = wid * SB

        pltpu.sync_copy(idx_hbm.at[:, pl.ds(b0w, SB)], idx_v)

        iota = lax.iota(jnp.int32, 16)

        def prep_fire(c, slot):
            s_ = c // NH
            boff = (c % NH) * TB
            for g in range(TB // 16):
                v = idx_v[s_, pl.ds(boff + g * 16, 16)]
                idx4_v[slot, pl.ds(g * 16, 16)] = v >> 2
                off_v[pl.ds(slot * TB + g * 16, 16)] = (v & 3) << 5
            pltpu.async_copy(
                w4_hbm.at[idx4_v.at[slot]],
                gbuf.at[pl.ds(slot * TB, TB)],
                gsem[slot],
            )

        def drain_gather(slot):
            pltpu.make_async_copy(
                w4_hbm.at[pl.ds(0, TB)],
                gbuf.at[pl.ds(slot * TB, TB)],
                gsem[slot],
            ).wait()

        def transpose(slot):
            gv = gbuf.at[pl.ds(slot * TB, TB)]
            vjs = [g * 16 + iota for g in range(TB // 16)]
            vcols = [off_v[pl.ds(slot * TB + g * 16, 16)]
                     for g in range(TB // 16)]

            def dbody(d, carry):
                for g in range(TB // 16):
                    v = plsc.load_gather(gv, [vjs[g], vcols[g] + d])
                    tbuf[slot * D + d, pl.ds(g * 16, 16)] = v
                return carry

            lax.fori_loop(0, D, dbody, 0)

        def writeback(c, slot):
            s_ = c // NH
            boff = (c % NH) * TB
            pltpu.async_copy(
                tbuf.at[pl.ds(slot * D, D)],
                out_hbm.at[s_, :, pl.ds(b0w + boff, TB)],
                wsem[slot],
            )

        def drain_wb(slot):
            pltpu.make_async_copy(
                tbuf.at[pl.ds(slot * D, D)],
                out_hbm.at[0, :, pl.ds(0, TB)],
                wsem[slot],
            ).wait()

        for p in range(NS - 1):
            prep_fire(p, p)

        def outer(c, carry):
            for slot in range(NS):
                cc = c + slot
                nxt = cc + NS - 1

                @pl.when(nxt < NT)
                def _():
                    prep_fire(nxt, (slot - 1) % NS)

                drain_gather(slot)

                @pl.when(cc >= NS)
                def _():
                    drain_wb(slot)

                transpose(slot)
                writeback(cc, slot)
            return carry

        lax.fori_loop(0, NT // NS, lambda i, cr: outer(i * NS, cr), 0)
        for p in range(NS):
            drain_wb(p)

    return body


def kernel(token_ids, weights):
    B0, S = token_ids.shape
    V, D = weights.shape
    idxp = jnp.pad(token_ids.T.astype(jnp.int32), ((0, (-S) % 8), (0, 0)))
    w4 = weights.reshape(V * D // 128, 128)
    oT = _build(S, D, B0)(idxp, w4)
    return oT.transpose(2, 0, 1)
